# baseline (speedup 1.0000x reference)
import functools

import jax
import jax.numpy as jnp
from jax import lax
from jax.experimental import pallas as pl
from jax.experimental.pallas import tpu as pltpu
from jax.experimental.pallas import tpu_sc as plsc

NC = 2
NS = 16
NW = NC * NS
L = 16

MAX_NORM = 1.0
EPS = 1e-7


def _rsqrt(x):
    i = plsc.bitcast(x, jnp.int32)
    i = jnp.int32(0x5F3759DF) - (i >> 1)
    y = plsc.bitcast(i, jnp.float32)
    for _ in range(3):
        y = y * (1.5 - 0.5 * x * y * y)
    return y


def _sqrt(x):
    xs = jnp.maximum(x, 1e-30)
    return xs * _rsqrt(xs)


def _gather_body(p_sub,
                 uidx_hbm, iidx_hbm, utab, itab, btab,
                 urows_out, irows_out, brows_out,
                 uidx_v, iidx_v, sem):
    wid = lax.axis_index("s") * NC + lax.axis_index("c")
    base = wid * p_sub

    pltpu.sync_copy(uidx_hbm.at[wid], uidx_v)
    pltpu.sync_copy(iidx_hbm.at[wid], iidx_v)

    def batch(g, _):
        uvec = uidx_v[pl.ds(g * L, L)]
        ivec = iidx_v[pl.ds(g * L, L)]
        for rr in range(L):
            iu = uvec[rr]
            ii = ivec[rr]
            r = base + g * L + rr
            dst = pl.ds(r, 1)
            pltpu.async_copy(utab.at[pl.ds(iu, 1)], urows_out.at[dst], sem)
            pltpu.async_copy(itab.at[pl.ds(ii, 1)], irows_out.at[dst], sem)
            pltpu.async_copy(btab.at[pl.ds(ii, 1)], brows_out.at[dst], sem)
        return 0

    lax.fori_loop(0, p_sub // L, batch, 0)

    sl = pl.ds(base, p_sub)
    pltpu.make_async_copy(utab.at[pl.ds(0, p_sub)], urows_out.at[sl], sem).wait()
    pltpu.make_async_copy(itab.at[pl.ds(0, p_sub)], irows_out.at[sl], sem).wait()
    pltpu.make_async_copy(btab.at[pl.ds(0, p_sub)], brows_out.at[sl], sem).wait()


def _compute_body(p_sub, emb,
                  urows_hbm, irows_hbm, brows_hbm,
                  pred_out, ul2_out, il2_out,
                  ubuf, ibuf, bbuf, pred_v, ul2_v, il2_v):
    wid = lax.axis_index("s") * NC + lax.axis_index("c")
    base = wid * p_sub
    chunk = ubuf.shape[0]

    pltpu.sync_copy(brows_hbm.at[pl.ds(base, p_sub)], bbuf)

    iota = lax.iota(jnp.int32, L)

    for j in range(p_sub // chunk):
        sl_in = pl.ds(base + j * chunk, chunk)
        pltpu.sync_copy(urows_hbm.at[sl_in], ubuf)
        pltpu.sync_copy(irows_hbm.at[sl_in], ibuf)

        def group(g, _, j=j):
            slot = iota + g * L
            acc_d = jnp.zeros((L,), jnp.float32)
            acc_u2 = jnp.zeros((L,), jnp.float32)
            acc_i2 = jnp.zeros((L,), jnp.float32)
            for e in range(emb):
                ecol = jnp.full((L,), e, jnp.int32)
                u_e = plsc.load_gather(ubuf, [slot, ecol])
                i_e = plsc.load_gather(ibuf, [slot, ecol])
                acc_d = acc_d + u_e * i_e
                acc_u2 = acc_u2 + u_e * u_e
                acc_i2 = acc_i2 + i_e * i_e
            bias = bbuf[pl.ds(j * chunk + g * L, L)]
            norm_u = _sqrt(acc_u2)
            norm_i = _sqrt(acc_i2)
            su = jnp.minimum(1.0, MAX_NORM / (norm_u + EPS))
            si = jnp.minimum(1.0, MAX_NORM / (norm_i + EPS))
            sl = pl.ds(j * chunk + g * L, L)
            pred_v[sl] = acc_d * (su * si) + bias
            ul2_v[sl] = norm_u * su
            il2_v[sl] = _sqrt(acc_i2 * (si * si) + bias * bias)
            return 0

        lax.fori_loop(0, chunk // L, group, 0)

    sl_out = pl.ds(base, p_sub)
    pltpu.sync_copy(pred_v, pred_out.at[sl_out])
    pltpu.sync_copy(ul2_v, ul2_out.at[sl_out])
    pltpu.sync_copy(il2_v, il2_out.at[sl_out])


def kernel(user_idx, item_i_idx, user_table, item_table, item_bias_table):
    b = user_idx.shape[0]
    emb = user_table.shape[1]
    p_sub = b // NW

    uidx2 = user_idx.astype(jnp.int32).reshape(NW, p_sub)
    iidx2 = item_i_idx.astype(jnp.int32).reshape(NW, p_sub)

    mesh = plsc.VectorSubcoreMesh(
        core_axis_name="c", subcore_axis_name="s",
        num_cores=NC, num_subcores=NS)

    f32 = jnp.float32
    i32 = jnp.int32

    urows, irows, brows = pl.kernel(
        functools.partial(_gather_body, p_sub),
        out_type=[
            jax.ShapeDtypeStruct((b, emb), f32),
            jax.ShapeDtypeStruct((b, emb), f32),
            jax.ShapeDtypeStruct((b, 1), f32),
        ],
        mesh=mesh,
        scratch_types=[
            pltpu.VMEM((p_sub,), i32),
            pltpu.VMEM((p_sub,), i32),
            pltpu.SemaphoreType.DMA,
        ],
    )(uidx2, iidx2, user_table, item_table, item_bias_table)

    pred, ul2, il2 = pl.kernel(
        functools.partial(_compute_body, p_sub, emb),
        out_type=[jax.ShapeDtypeStruct((b,), f32)] * 3,
        mesh=mesh,
        compiler_params=pltpu.CompilerParams(needs_layout_passes=False),
        scratch_types=[
            pltpu.VMEM((128, emb), f32),
            pltpu.VMEM((128, emb), f32),
            pltpu.VMEM((p_sub,), f32),
            pltpu.VMEM((p_sub,), f32),
            pltpu.VMEM((p_sub,), f32),
            pltpu.VMEM((p_sub,), f32),
        ],
    )(urows, irows, brows.reshape(b))
    return pred.reshape(b, 1), ul2, il2


# speedup vs baseline: 1.8677x; 1.8677x over previous
import functools

import jax
import jax.numpy as jnp
from jax import lax
from jax.experimental import pallas as pl
from jax.experimental.pallas import tpu as pltpu
from jax.experimental.pallas import tpu_sc as plsc

NC = 2
NS = 16
NW = NC * NS
L = 16

MAX_NORM = 1.0
EPS = 1e-7


def _rsqrt(x):
    i = plsc.bitcast(x, jnp.int32)
    i = jnp.int32(0x5F3759DF) - (i >> 1)
    y = plsc.bitcast(i, jnp.float32)
    for _ in range(3):
        y = y * (1.5 - 0.5 * x * y * y)
    return y


def _sqrt(x):
    xs = jnp.maximum(x, 1e-30)
    return xs * _rsqrt(xs)


def _gather_body(p_sub, chunk,
                 uidx_hbm, iidx_hbm, utab, itab, btab,
                 urows_out, irows_out, brows_out,
                 uidx_v, iidx_v, ub0, ib0, bb0, ub1, ib1, bb1,
                 sem0, sem1):
    wid = lax.axis_index("s") * NC + lax.axis_index("c")
    base = wid * p_sub
    n_chunks = p_sub // chunk

    pltpu.sync_copy(uidx_hbm.at[wid], uidx_v)
    pltpu.sync_copy(iidx_hbm.at[wid], iidx_v)

    bufs = [(ub0, ib0, bb0, sem0), (ub1, ib1, bb1, sem1)]

    def fire(j, ub, ib, bb, sem):
        def batch(g, _):
            uvec = uidx_v[pl.ds(j * chunk + g * L, L)]
            ivec = iidx_v[pl.ds(j * chunk + g * L, L)]
            for rr in range(L):
                iu = uvec[rr]
                ii = ivec[rr]
                dst = pl.ds(g * L + rr, 1)
                pltpu.async_copy(utab.at[pl.ds(iu, 1)], ub.at[dst], sem)
                pltpu.async_copy(itab.at[pl.ds(ii, 1)], ib.at[dst], sem)
                pltpu.async_copy(btab.at[pl.ds(ii, 1)], bb.at[dst], sem)
            return 0
        lax.fori_loop(0, chunk // L, batch, 0)

    fire(0, *bufs[0])
    for j in range(n_chunks):
        ub, ib, bb, sem = bufs[j % 2]
        if j + 1 < n_chunks:
            fire(j + 1, *bufs[(j + 1) % 2])
        pltpu.make_async_copy(utab.at[pl.ds(0, chunk)], ub, sem).wait()
        pltpu.make_async_copy(itab.at[pl.ds(0, chunk)], ib, sem).wait()
        pltpu.make_async_copy(btab.at[pl.ds(0, chunk)], bb, sem).wait()
        sl = pl.ds(base + j * chunk, chunk)
        pltpu.sync_copy(ub, urows_out.at[sl])
        pltpu.sync_copy(ib, irows_out.at[sl])
        pltpu.sync_copy(bb, brows_out.at[sl])


def _compute_body(p_sub, emb,
                  urows_hbm, irows_hbm, brows_hbm,
                  pred_out, ul2_out, il2_out,
                  ubuf, ibuf, bbuf, pred_v, ul2_v, il2_v):
    wid = lax.axis_index("s") * NC + lax.axis_index("c")
    base = wid * p_sub
    chunk = ubuf.shape[0]

    pltpu.sync_copy(brows_hbm.at[pl.ds(base, p_sub)], bbuf)

    iota = lax.iota(jnp.int32, L)

    for j in range(p_sub // chunk):
        sl_in = pl.ds(base + j * chunk, chunk)
        pltpu.sync_copy(urows_hbm.at[sl_in], ubuf)
        pltpu.sync_copy(irows_hbm.at[sl_in], ibuf)

        def group(g, _, j=j):
            slot = iota + g * L
            acc_d = jnp.zeros((L,), jnp.float32)
            acc_u2 = jnp.zeros((L,), jnp.float32)
            acc_i2 = jnp.zeros((L,), jnp.float32)
            for e in range(emb):
                ecol = jnp.full((L,), e, jnp.int32)
                u_e = plsc.load_gather(ubuf, [slot, ecol])
                i_e = plsc.load_gather(ibuf, [slot, ecol])
                acc_d = acc_d + u_e * i_e
                acc_u2 = acc_u2 + u_e * u_e
                acc_i2 = acc_i2 + i_e * i_e
            bias = bbuf[pl.ds(j * chunk + g * L, L)]
            norm_u = _sqrt(acc_u2)
            norm_i = _sqrt(acc_i2)
            su = jnp.minimum(1.0, MAX_NORM / (norm_u + EPS))
            si = jnp.minimum(1.0, MAX_NORM / (norm_i + EPS))
            sl = pl.ds(j * chunk + g * L, L)
            pred_v[sl] = acc_d * (su * si) + bias
            ul2_v[sl] = norm_u * su
            il2_v[sl] = _sqrt(acc_i2 * (si * si) + bias * bias)
            return 0

        lax.fori_loop(0, chunk // L, group, 0)

    sl_out = pl.ds(base, p_sub)
    pltpu.sync_copy(pred_v, pred_out.at[sl_out])
    pltpu.sync_copy(ul2_v, ul2_out.at[sl_out])
    pltpu.sync_copy(il2_v, il2_out.at[sl_out])


def kernel(user_idx, item_i_idx, user_table, item_table, item_bias_table):
    b = user_idx.shape[0]
    emb = user_table.shape[1]
    p_sub = b // NW

    uidx2 = user_idx.astype(jnp.int32).reshape(NW, p_sub)
    iidx2 = item_i_idx.astype(jnp.int32).reshape(NW, p_sub)

    mesh = plsc.VectorSubcoreMesh(
        core_axis_name="c", subcore_axis_name="s",
        num_cores=NC, num_subcores=NS)

    f32 = jnp.float32
    i32 = jnp.int32

    chunk = 128
    urows, irows, brows = pl.kernel(
        functools.partial(_gather_body, p_sub, chunk),
        out_type=[
            jax.ShapeDtypeStruct((b, emb), f32),
            jax.ShapeDtypeStruct((b, emb), f32),
            jax.ShapeDtypeStruct((b, 1), f32),
        ],
        mesh=mesh,
        scratch_types=[
            pltpu.VMEM((p_sub,), i32),
            pltpu.VMEM((p_sub,), i32),
            pltpu.VMEM((chunk, emb), f32),
            pltpu.VMEM((chunk, emb), f32),
            pltpu.VMEM((chunk, 1), f32),
            pltpu.VMEM((chunk, emb), f32),
            pltpu.VMEM((chunk, emb), f32),
            pltpu.VMEM((chunk, 1), f32),
            pltpu.SemaphoreType.DMA,
            pltpu.SemaphoreType.DMA,
        ],
    )(uidx2, iidx2, user_table, item_table, item_bias_table)

    pred, ul2, il2 = pl.kernel(
        functools.partial(_compute_body, p_sub, emb),
        out_type=[jax.ShapeDtypeStruct((b,), f32)] * 3,
        mesh=mesh,
        compiler_params=pltpu.CompilerParams(needs_layout_passes=False),
        scratch_types=[
            pltpu.VMEM((128, emb), f32),
            pltpu.VMEM((128, emb), f32),
            pltpu.VMEM((p_sub,), f32),
            pltpu.VMEM((p_sub,), f32),
            pltpu.VMEM((p_sub,), f32),
            pltpu.VMEM((p_sub,), f32),
        ],
    )(urows, irows, brows.reshape(b))
    return pred.reshape(b, 1), ul2, il2


# speedup vs baseline: 1.8681x; 1.0002x over previous
import functools

import jax
import jax.numpy as jnp
from jax import lax
from jax.experimental import pallas as pl
from jax.experimental.pallas import tpu as pltpu
from jax.experimental.pallas import tpu_sc as plsc

NC = 2
NS = 16
NW = NC * NS
L = 16

MAX_NORM = 1.0
EPS = 1e-7


def _rsqrt(x):
    i = plsc.bitcast(x, jnp.int32)
    i = jnp.int32(0x5F3759DF) - (i >> 1)
    y = plsc.bitcast(i, jnp.float32)
    for _ in range(3):
        y = y * (1.5 - 0.5 * x * y * y)
    return y


def _sqrt(x):
    xs = jnp.maximum(x, 1e-30)
    return xs * _rsqrt(xs)


def _gather_body(p_sub, chunk,
                 uidx_hbm, iidx_hbm, utab, itab, btab,
                 urows_out, irows_out, brows_out,
                 uidx_v, iidx_v, ub0, ib0, bb0, ub1, ib1, bb1,
                 sem0, sem1):
    wid = lax.axis_index("s") * NC + lax.axis_index("c")
    base = wid * p_sub
    n_chunks = p_sub // chunk

    pltpu.sync_copy(uidx_hbm.at[wid], uidx_v)
    pltpu.sync_copy(iidx_hbm.at[wid], iidx_v)

    bufs = [(ub0, ib0, bb0, sem0), (ub1, ib1, bb1, sem1)]

    def fire(j, ub, ib, bb, sem):
        def batch(g, _):
            uvec = uidx_v[pl.ds(j * chunk + g * L, L)]
            ivec = iidx_v[pl.ds(j * chunk + g * L, L)]
            for rr in range(L):
                iu = uvec[rr]
                ii = ivec[rr]
                dst = pl.ds(g * L + rr, 1)
                pltpu.async_copy(utab.at[pl.ds(iu, 1)], ub.at[dst], sem)
                pltpu.async_copy(itab.at[pl.ds(ii, 1)], ib.at[dst], sem)
                pltpu.async_copy(btab.at[pl.ds(ii, 1)], bb.at[dst], sem)
            return 0
        lax.fori_loop(0, chunk // L, batch, 0)

    fire(0, *bufs[0])
    for j in range(n_chunks):
        ub, ib, bb, sem = bufs[j % 2]
        if j + 1 < n_chunks:
            fire(j + 1, *bufs[(j + 1) % 2])
        pltpu.make_async_copy(utab.at[pl.ds(0, chunk)], ub, sem).wait()
        pltpu.make_async_copy(itab.at[pl.ds(0, chunk)], ib, sem).wait()
        pltpu.make_async_copy(btab.at[pl.ds(0, chunk)], bb, sem).wait()
        sl = pl.ds(base + j * chunk, chunk)
        pltpu.sync_copy(ub, urows_out.at[sl])
        pltpu.sync_copy(ib, irows_out.at[sl])
        pltpu.sync_copy(bb, brows_out.at[sl])


def _compute_body(p_sub, emb,
                  urows_hbm, irows_hbm, brows_hbm,
                  pred_out, ul2_out, il2_out,
                  ubuf, ibuf, bbuf, pred_v, ul2_v, il2_v):
    wid = lax.axis_index("s") * NC + lax.axis_index("c")
    base = wid * p_sub
    chunk = ubuf.shape[0]

    pltpu.sync_copy(brows_hbm.at[pl.ds(base, p_sub)], bbuf)

    iota = lax.iota(jnp.int32, L)

    for j in range(p_sub // chunk):
        sl_in = pl.ds(base + j * chunk, chunk)
        pltpu.sync_copy(urows_hbm.at[sl_in], ubuf)
        pltpu.sync_copy(irows_hbm.at[sl_in], ibuf)

        def group(g, _, j=j):
            slot = iota + g * L
            acc_d = jnp.zeros((L,), jnp.float32)
            acc_u2 = jnp.zeros((L,), jnp.float32)
            acc_i2 = jnp.zeros((L,), jnp.float32)
            for e in range(emb):
                ecol = jnp.full((L,), e, jnp.int32)
                u_e = plsc.load_gather(ubuf, [slot, ecol])
                i_e = plsc.load_gather(ibuf, [slot, ecol])
                acc_d = acc_d + u_e * i_e
                acc_u2 = acc_u2 + u_e * u_e
                acc_i2 = acc_i2 + i_e * i_e
            bias = bbuf[pl.ds(j * chunk + g * L, L)]
            norm_u = _sqrt(acc_u2)
            norm_i = _sqrt(acc_i2)
            su = jnp.minimum(1.0, MAX_NORM / (norm_u + EPS))
            si = jnp.minimum(1.0, MAX_NORM / (norm_i + EPS))
            sl = pl.ds(j * chunk + g * L, L)
            pred_v[sl] = acc_d * (su * si) + bias
            ul2_v[sl] = norm_u * su
            il2_v[sl] = _sqrt(acc_i2 * (si * si) + bias * bias)
            return 0

        lax.fori_loop(0, chunk // L, group, 0)

    sl_out = pl.ds(base, p_sub)
    pltpu.sync_copy(pred_v, pred_out.at[sl_out])
    pltpu.sync_copy(ul2_v, ul2_out.at[sl_out])
    pltpu.sync_copy(il2_v, il2_out.at[sl_out])


def kernel(user_idx, item_i_idx, user_table, item_table, item_bias_table):
    b = user_idx.shape[0]
    emb = user_table.shape[1]
    p_sub = b // NW

    uidx2 = user_idx.astype(jnp.int32).reshape(NW, p_sub)
    iidx2 = item_i_idx.astype(jnp.int32).reshape(NW, p_sub)

    mesh = plsc.VectorSubcoreMesh(
        core_axis_name="c", subcore_axis_name="s",
        num_cores=NC, num_subcores=NS)

    f32 = jnp.float32
    i32 = jnp.int32

    chunk = 128
    urows, irows, brows = pl.kernel(
        functools.partial(_gather_body, p_sub, chunk),
        out_type=[
            jax.ShapeDtypeStruct((b, emb), f32),
            jax.ShapeDtypeStruct((b, emb), f32),
            jax.ShapeDtypeStruct((b, 1), f32),
        ],
        mesh=mesh,
        compiler_params=pltpu.CompilerParams(use_tc_tiling_on_sc=True),
        scratch_types=[
            pltpu.VMEM((p_sub,), i32),
            pltpu.VMEM((p_sub,), i32),
            pltpu.VMEM((chunk, emb), f32),
            pltpu.VMEM((chunk, emb), f32),
            pltpu.VMEM((chunk, 1), f32),
            pltpu.VMEM((chunk, emb), f32),
            pltpu.VMEM((chunk, emb), f32),
            pltpu.VMEM((chunk, 1), f32),
            pltpu.SemaphoreType.DMA,
            pltpu.SemaphoreType.DMA,
        ],
    )(uidx2, iidx2, user_table, item_table, item_bias_table)

    pred, ul2, il2 = pl.kernel(
        functools.partial(_compute_body, p_sub, emb),
        out_type=[jax.ShapeDtypeStruct((b,), f32)] * 3,
        mesh=mesh,
        compiler_params=pltpu.CompilerParams(needs_layout_passes=False),
        scratch_types=[
            pltpu.VMEM((128, emb), f32),
            pltpu.VMEM((128, emb), f32),
            pltpu.VMEM((p_sub,), f32),
            pltpu.VMEM((p_sub,), f32),
            pltpu.VMEM((p_sub,), f32),
            pltpu.VMEM((p_sub,), f32),
        ],
    )(urows, irows, brows.reshape(b))
    return pred.reshape(b, 1), ul2, il2
